# Initial kernel scaffold; baseline (speedup 1.0000x reference)
import jax
import jax.numpy as jnp
from jax.experimental import pallas as pl


def kernel(x, i):
    raise NotImplementedError("write your pallas kernel here")


# speedup vs baseline: 8.0382x; 8.0382x over previous
import jax
import jax.numpy as jnp
from jax.experimental import pallas as pl
from jax.experimental.pallas import tpu as pltpu

_N = 320000
_F = 128
_S = 512
_EPS = 0.001
_B = 2560
_NB = _N // _B
_W = 32
_SPAD = _S + _W


def _stats_kernel(x_ref, seg_ref, table_ref, acc_sum, acc_aux):
    b = pl.program_id(0)

    @pl.when(b == 0)
    def _init():
        acc_sum[...] = jnp.zeros_like(acc_sum)
        acc_aux[...] = jnp.zeros_like(acc_aux)

    seg = seg_ref[0, 0, :]
    s0 = (seg[0] // 8) * 8
    smax = seg[_B - 1]
    x = x_ref[...]
    rowssq = jnp.sum(x * x, axis=1, keepdims=True)
    aux_in = jnp.concatenate([rowssq, jnp.ones_like(rowssq)], axis=1)

    narrow = (smax - s0) < _W

    @pl.when(narrow)
    def _narrow():
        col = jax.lax.broadcasted_iota(jnp.int32, (_B, _W), 1)
        oh = (seg[:, None] - s0 == col).astype(jnp.float32)
        ps = jax.lax.dot_general(
            oh, x, (((0,), (0,)), ((), ())),
            preferred_element_type=jnp.float32,
            precision=jax.lax.Precision.HIGHEST)
        pa = jax.lax.dot_general(
            oh, aux_in, (((0,), (0,)), ((), ())),
            preferred_element_type=jnp.float32,
            precision=jax.lax.Precision.HIGHEST)
        acc_sum[pl.ds(s0, _W), :] += ps
        acc_aux[pl.ds(s0, _W), :] += pa

    @pl.when(jnp.logical_not(narrow))
    def _wide():
        col = jax.lax.broadcasted_iota(jnp.int32, (_B, _S), 1)
        oh = (seg[:, None] == col).astype(jnp.float32)
        ps = jax.lax.dot_general(
            oh, x, (((0,), (0,)), ((), ())),
            preferred_element_type=jnp.float32,
            precision=jax.lax.Precision.HIGHEST)
        pa = jax.lax.dot_general(
            oh, aux_in, (((0,), (0,)), ((), ())),
            preferred_element_type=jnp.float32,
            precision=jax.lax.Precision.HIGHEST)
        acc_sum[pl.ds(0, _S), :] += ps
        acc_aux[pl.ds(0, _S), :] += pa

    @pl.when(b == _NB - 1)
    def _finalize():
        cnt = acc_aux[:, 1:2]
        ssq_tot = acc_aux[:, 0:1]
        mean = acc_sum[...] / jnp.maximum(cnt, 1.0)
        ssq = ssq_tot - cnt * jnp.sum(mean * mean, axis=1, keepdims=True)
        var = ssq / (cnt * jnp.float32(_F) - 1.0)
        invstd = jax.lax.rsqrt(var + _EPS)
        table_ref[:, 0:_F] = mean * invstd
        table_ref[:, _F:2 * _F] = jnp.broadcast_to(invstd, (_SPAD, _F))


def _norm_kernel(x_ref, seg_ref, table_ref, out_ref):
    seg = seg_ref[0, 0, :]
    s0 = (seg[0] // 8) * 8
    smax = seg[_B - 1]
    x = x_ref[...]

    narrow = (smax - s0) < _W

    @pl.when(narrow)
    def _narrow():
        win = table_ref[pl.ds(s0, _W), :]
        col = jax.lax.broadcasted_iota(jnp.int32, (_B, _W), 1)
        oh = (seg[:, None] - s0 == col).astype(jnp.float32)
        rows = jax.lax.dot_general(
            oh, win, (((1,), (0,)), ((), ())),
            preferred_element_type=jnp.float32,
            precision=jax.lax.Precision.HIGHEST)
        out_ref[...] = x * rows[:, _F:2 * _F] - rows[:, 0:_F]

    @pl.when(jnp.logical_not(narrow))
    def _wide():
        win = table_ref[pl.ds(0, _S), :]
        col = jax.lax.broadcasted_iota(jnp.int32, (_B, _S), 1)
        oh = (seg[:, None] == col).astype(jnp.float32)
        rows = jax.lax.dot_general(
            oh, win, (((1,), (0,)), ((), ())),
            preferred_element_type=jnp.float32,
            precision=jax.lax.Precision.HIGHEST)
        out_ref[...] = x * rows[:, _F:2 * _F] - rows[:, 0:_F]


def kernel(x, i):
    seg = i.astype(jnp.int32)
    seg3 = seg.reshape(_NB, 1, _B)

    table = pl.pallas_call(
        _stats_kernel,
        grid=(_NB,),
        in_specs=[
            pl.BlockSpec((_B, _F), lambda b: (b, 0)),
            pl.BlockSpec((1, 1, _B), lambda b: (b, 0, 0)),
        ],
        out_specs=pl.BlockSpec((_SPAD, 2 * _F), lambda b: (0, 0)),
        out_shape=jax.ShapeDtypeStruct((_SPAD, 2 * _F), jnp.float32),
        scratch_shapes=[
            pltpu.VMEM((_SPAD, _F), jnp.float32),
            pltpu.VMEM((_SPAD, 2), jnp.float32),
        ],
    )(x, seg3)

    out = pl.pallas_call(
        _norm_kernel,
        grid=(_NB,),
        in_specs=[
            pl.BlockSpec((_B, _F), lambda b: (b, 0)),
            pl.BlockSpec((1, 1, _B), lambda b: (b, 0, 0)),
            pl.BlockSpec((_SPAD, 2 * _F), lambda b: (0, 0)),
        ],
        out_specs=pl.BlockSpec((_B, _F), lambda b: (b, 0)),
        out_shape=jax.ShapeDtypeStruct((_N, _F), jnp.float32),
    )(x, seg3, table)

    return out


# speedup vs baseline: 14.2715x; 1.7754x over previous
import jax
import jax.numpy as jnp
from jax.experimental import pallas as pl
from jax.experimental.pallas import tpu as pltpu

_N = 320000
_F = 128
_S = 512
_EPS = 0.001
_B = 2560
_NB = _N // _B
_W = 32
_SPAD = _S + _W


def _stats_kernel(x_ref, seg_ref, table_ref, acc_sum, acc_aux):
    b = pl.program_id(0)

    @pl.when(b == 0)
    def _init():
        acc_sum[...] = jnp.zeros_like(acc_sum)
        acc_aux[...] = jnp.zeros_like(acc_aux)

    seg = seg_ref[0, 0, :]
    s0 = (seg[0] // 8) * 8
    smax = seg[_B - 1]
    x = x_ref[...]
    rowssq = jnp.sum(x * x, axis=1, keepdims=True)
    aux_in = jnp.concatenate([rowssq, jnp.ones_like(rowssq)], axis=1)

    narrow = (smax - s0) < _W

    @pl.when(narrow)
    def _narrow():
        col = jax.lax.broadcasted_iota(jnp.int32, (_B, _W), 1)
        oh = (seg[:, None] - s0 == col).astype(jnp.float32)
        ps = jax.lax.dot_general(
            oh, x, (((0,), (0,)), ((), ())),
            preferred_element_type=jnp.float32,
            precision=jax.lax.Precision.DEFAULT)
        pa = jax.lax.dot_general(
            oh, aux_in, (((0,), (0,)), ((), ())),
            preferred_element_type=jnp.float32,
            precision=jax.lax.Precision.DEFAULT)
        acc_sum[pl.ds(s0, _W), :] += ps
        acc_aux[pl.ds(s0, _W), :] += pa

    @pl.when(jnp.logical_not(narrow))
    def _wide():
        col = jax.lax.broadcasted_iota(jnp.int32, (_B, _S), 1)
        oh = (seg[:, None] == col).astype(jnp.float32)
        ps = jax.lax.dot_general(
            oh, x, (((0,), (0,)), ((), ())),
            preferred_element_type=jnp.float32,
            precision=jax.lax.Precision.DEFAULT)
        pa = jax.lax.dot_general(
            oh, aux_in, (((0,), (0,)), ((), ())),
            preferred_element_type=jnp.float32,
            precision=jax.lax.Precision.DEFAULT)
        acc_sum[pl.ds(0, _S), :] += ps
        acc_aux[pl.ds(0, _S), :] += pa

    @pl.when(b == _NB - 1)
    def _finalize():
        cnt = acc_aux[:, 1:2]
        ssq_tot = acc_aux[:, 0:1]
        mean = acc_sum[...] / jnp.maximum(cnt, 1.0)
        ssq = ssq_tot - cnt * jnp.sum(mean * mean, axis=1, keepdims=True)
        var = ssq / (cnt * jnp.float32(_F) - 1.0)
        invstd = jax.lax.rsqrt(var + _EPS)
        table_ref[:, 0:_F] = mean * invstd
        table_ref[:, _F:2 * _F] = jnp.broadcast_to(invstd, (_SPAD, _F))


def _norm_kernel(x_ref, seg_ref, table_ref, out_ref):
    seg = seg_ref[0, 0, :]
    s0 = (seg[0] // 8) * 8
    smax = seg[_B - 1]
    x = x_ref[...]

    narrow = (smax - s0) < _W

    @pl.when(narrow)
    def _narrow():
        win = table_ref[pl.ds(s0, _W), :]
        col = jax.lax.broadcasted_iota(jnp.int32, (_B, _W), 1)
        oh = (seg[:, None] - s0 == col).astype(jnp.float32)
        rows = jax.lax.dot_general(
            oh, win, (((1,), (0,)), ((), ())),
            preferred_element_type=jnp.float32,
            precision=jax.lax.Precision.DEFAULT)
        out_ref[...] = x * rows[:, _F:2 * _F] - rows[:, 0:_F]

    @pl.when(jnp.logical_not(narrow))
    def _wide():
        win = table_ref[pl.ds(0, _S), :]
        col = jax.lax.broadcasted_iota(jnp.int32, (_B, _S), 1)
        oh = (seg[:, None] == col).astype(jnp.float32)
        rows = jax.lax.dot_general(
            oh, win, (((1,), (0,)), ((), ())),
            preferred_element_type=jnp.float32,
            precision=jax.lax.Precision.DEFAULT)
        out_ref[...] = x * rows[:, _F:2 * _F] - rows[:, 0:_F]


def kernel(x, i):
    seg = i.astype(jnp.int32)
    seg3 = seg.reshape(_NB, 1, _B)

    table = pl.pallas_call(
        _stats_kernel,
        grid=(_NB,),
        in_specs=[
            pl.BlockSpec((_B, _F), lambda b: (b, 0)),
            pl.BlockSpec((1, 1, _B), lambda b: (b, 0, 0)),
        ],
        out_specs=pl.BlockSpec((_SPAD, 2 * _F), lambda b: (0, 0)),
        out_shape=jax.ShapeDtypeStruct((_SPAD, 2 * _F), jnp.float32),
        scratch_shapes=[
            pltpu.VMEM((_SPAD, _F), jnp.float32),
            pltpu.VMEM((_SPAD, 2), jnp.float32),
        ],
    )(x, seg3)

    out = pl.pallas_call(
        _norm_kernel,
        grid=(_NB,),
        in_specs=[
            pl.BlockSpec((_B, _F), lambda b: (b, 0)),
            pl.BlockSpec((1, 1, _B), lambda b: (b, 0, 0)),
            pl.BlockSpec((_SPAD, 2 * _F), lambda b: (0, 0)),
        ],
        out_specs=pl.BlockSpec((_B, _F), lambda b: (b, 0)),
        out_shape=jax.ShapeDtypeStruct((_N, _F), jnp.float32),
    )(x, seg3, table)

    return out


# speedup vs baseline: 17.5521x; 1.2299x over previous
import jax
import jax.numpy as jnp
from jax.experimental import pallas as pl
from jax.experimental.pallas import tpu as pltpu

_N = 320000
_F = 128
_S = 512
_EPS = 0.001
_B = 6400
_NB = _N // _B
_W = 32
_SPAD = _S + _W


def _partial_stats(oh_bf, x):
    rowssq = jnp.sum(x * x, axis=1, keepdims=True)
    z = jnp.concatenate([x, rowssq, jnp.ones_like(rowssq)], axis=1)
    z_bf = z.astype(jnp.bfloat16)
    return jax.lax.dot_general(
        oh_bf, z_bf, (((0,), (0,)), ((), ())),
        preferred_element_type=jnp.float32)


def _stats_kernel(x_ref, seg_ref, table_ref, acc_ref):
    b = pl.program_id(0)

    @pl.when(b == 0)
    def _init():
        acc_ref[...] = jnp.zeros_like(acc_ref)

    seg = seg_ref[0, 0, :]
    s0 = (seg[0] // 8) * 8
    smax = seg[_B - 1]
    x = x_ref[...]

    narrow = (smax - s0) < _W

    @pl.when(narrow)
    def _narrow():
        col = jax.lax.broadcasted_iota(jnp.int32, (_B, _W), 1)
        oh = (seg[:, None] - s0 == col).astype(
            jnp.float32).astype(jnp.bfloat16)
        acc_ref[pl.ds(s0, _W), :] += _partial_stats(oh, x)

    @pl.when(jnp.logical_not(narrow))
    def _wide():
        col = jax.lax.broadcasted_iota(jnp.int32, (_B, _S), 1)
        oh = (seg[:, None] == col).astype(
            jnp.float32).astype(jnp.bfloat16)
        acc_ref[pl.ds(0, _S), :] += _partial_stats(oh, x)

    @pl.when(b == _NB - 1)
    def _finalize():
        ssq_tot = acc_ref[:, _F:_F + 1]
        cnt = acc_ref[:, _F + 1:_F + 2]
        mean = acc_ref[:, 0:_F] / jnp.maximum(cnt, 1.0)
        ssq = ssq_tot - cnt * jnp.sum(mean * mean, axis=1, keepdims=True)
        var = ssq / (cnt * jnp.float32(_F) - 1.0)
        invstd = jax.lax.rsqrt(var + _EPS)
        table_ref[:, 0:_F] = mean * invstd
        table_ref[:, _F:2 * _F] = jnp.broadcast_to(invstd, (_SPAD, _F))


def _apply_rows(oh_bf, win, x, out_ref):
    win_hi = win.astype(jnp.bfloat16)
    win_lo = (win - win_hi.astype(jnp.float32)).astype(jnp.bfloat16)
    dims = (((1,), (0,)), ((), ()))
    rows = (jax.lax.dot_general(oh_bf, win_hi, dims,
                                preferred_element_type=jnp.float32)
            + jax.lax.dot_general(oh_bf, win_lo, dims,
                                  preferred_element_type=jnp.float32))
    out_ref[...] = x * rows[:, _F:2 * _F] - rows[:, 0:_F]


def _norm_kernel(x_ref, seg_ref, table_ref, out_ref):
    seg = seg_ref[0, 0, :]
    s0 = (seg[0] // 8) * 8
    smax = seg[_B - 1]
    x = x_ref[...]

    narrow = (smax - s0) < _W

    @pl.when(narrow)
    def _narrow():
        win = table_ref[pl.ds(s0, _W), :]
        col = jax.lax.broadcasted_iota(jnp.int32, (_B, _W), 1)
        oh = (seg[:, None] - s0 == col).astype(
            jnp.float32).astype(jnp.bfloat16)
        _apply_rows(oh, win, x, out_ref)

    @pl.when(jnp.logical_not(narrow))
    def _wide():
        win = table_ref[pl.ds(0, _S), :]
        col = jax.lax.broadcasted_iota(jnp.int32, (_B, _S), 1)
        oh = (seg[:, None] == col).astype(
            jnp.float32).astype(jnp.bfloat16)
        _apply_rows(oh, win, x, out_ref)


def kernel(x, i):
    seg = i.astype(jnp.int32)
    seg3 = seg.reshape(_NB, 1, _B)

    table = pl.pallas_call(
        _stats_kernel,
        grid=(_NB,),
        in_specs=[
            pl.BlockSpec((_B, _F), lambda b: (b, 0)),
            pl.BlockSpec((1, 1, _B), lambda b: (b, 0, 0)),
        ],
        out_specs=pl.BlockSpec((_SPAD, 2 * _F), lambda b: (0, 0)),
        out_shape=jax.ShapeDtypeStruct((_SPAD, 2 * _F), jnp.float32),
        scratch_shapes=[
            pltpu.VMEM((_SPAD, _F + 2), jnp.float32),
        ],
    )(x, seg3)

    out = pl.pallas_call(
        _norm_kernel,
        grid=(_NB,),
        in_specs=[
            pl.BlockSpec((_B, _F), lambda b: (b, 0)),
            pl.BlockSpec((1, 1, _B), lambda b: (b, 0, 0)),
            pl.BlockSpec((_SPAD, 2 * _F), lambda b: (0, 0)),
        ],
        out_specs=pl.BlockSpec((_B, _F), lambda b: (b, 0)),
        out_shape=jax.ShapeDtypeStruct((_N, _F), jnp.float32),
    )(x, seg3, table)

    return out


# speedup vs baseline: 21.5650x; 1.2286x over previous
import jax
import jax.numpy as jnp
from jax.experimental import pallas as pl
from jax.experimental.pallas import tpu as pltpu

_N = 320000
_F = 128
_S = 512
_EPS = 0.001
_B = 6400
_NB = _N // _B
_W = 32
_SPAD = _S + _W


def _accumulate(oh_bf, oh_f32, x_bf, s0, w, acc_sum, acc_sq, acc_cnt):
    dims = (((0,), (0,)), ((), ()))
    ps = jax.lax.dot_general(oh_bf, x_bf, dims,
                             preferred_element_type=jnp.float32)
    psq = jax.lax.dot_general(oh_bf, x_bf * x_bf, dims,
                              preferred_element_type=jnp.float32)
    cnt = jnp.sum(oh_f32, axis=0).reshape(w, 1)
    acc_sum[pl.ds(s0, w), :] += ps
    acc_sq[pl.ds(s0, w), :] += psq
    acc_cnt[pl.ds(s0, w), :] += cnt


def _stats_kernel(x_ref, seg_ref, table_ref, acc_sum, acc_sq, acc_cnt):
    b = pl.program_id(0)

    @pl.when(b == 0)
    def _init():
        acc_sum[...] = jnp.zeros_like(acc_sum)
        acc_sq[...] = jnp.zeros_like(acc_sq)
        acc_cnt[...] = jnp.zeros_like(acc_cnt)

    seg = seg_ref[0, 0, :]
    s0 = (seg[0] // 8) * 8
    smax = seg[_B - 1]
    x_bf = x_ref[...].astype(jnp.bfloat16)

    narrow = (smax - s0) < _W

    @pl.when(narrow)
    def _narrow():
        col = jax.lax.broadcasted_iota(jnp.int32, (_B, _W), 1)
        oh_f32 = (seg[:, None] - s0 == col).astype(jnp.float32)
        _accumulate(oh_f32.astype(jnp.bfloat16), oh_f32, x_bf,
                    s0, _W, acc_sum, acc_sq, acc_cnt)

    @pl.when(jnp.logical_not(narrow))
    def _wide():
        col = jax.lax.broadcasted_iota(jnp.int32, (_B, _S), 1)
        oh_f32 = (seg[:, None] == col).astype(jnp.float32)
        _accumulate(oh_f32.astype(jnp.bfloat16), oh_f32, x_bf,
                    0, _S, acc_sum, acc_sq, acc_cnt)

    @pl.when(b == _NB - 1)
    def _finalize():
        cnt = acc_cnt[...]
        mean = acc_sum[...] / jnp.maximum(cnt, 1.0)
        ssq = (jnp.sum(acc_sq[...], axis=1, keepdims=True)
               - cnt * jnp.sum(mean * mean, axis=1, keepdims=True))
        var = ssq / (cnt * jnp.float32(_F) - 1.0)
        invstd = jax.lax.rsqrt(var + _EPS)
        table_ref[:, 0:_F] = mean.astype(jnp.bfloat16)
        table_ref[:, _F:2 * _F] = jnp.broadcast_to(
            invstd, (_SPAD, _F)).astype(jnp.bfloat16)


def _norm_kernel(x_ref, seg_ref, table_ref, out_ref):
    seg = seg_ref[0, 0, :]
    s0 = (seg[0] // 16) * 16
    smax = seg[_B - 1]
    x = x_ref[...]

    narrow = (smax - s0) < _W

    @pl.when(narrow)
    def _narrow():
        win = table_ref[pl.ds(s0, _W), :]
        col = jax.lax.broadcasted_iota(jnp.int32, (_B, _W), 1)
        oh = (seg[:, None] - s0 == col).astype(
            jnp.float32).astype(jnp.bfloat16)
        rows = jax.lax.dot_general(
            oh, win, (((1,), (0,)), ((), ())),
            preferred_element_type=jnp.float32)
        out_ref[...] = (x - rows[:, 0:_F]) * rows[:, _F:2 * _F]

    @pl.when(jnp.logical_not(narrow))
    def _wide():
        win = table_ref[pl.ds(0, _S), :]
        col = jax.lax.broadcasted_iota(jnp.int32, (_B, _S), 1)
        oh = (seg[:, None] == col).astype(
            jnp.float32).astype(jnp.bfloat16)
        rows = jax.lax.dot_general(
            oh, win, (((1,), (0,)), ((), ())),
            preferred_element_type=jnp.float32)
        out_ref[...] = (x - rows[:, 0:_F]) * rows[:, _F:2 * _F]


def kernel(x, i):
    seg = i.astype(jnp.int32)
    seg3 = seg.reshape(_NB, 1, _B)

    table = pl.pallas_call(
        _stats_kernel,
        grid=(_NB,),
        in_specs=[
            pl.BlockSpec((_B, _F), lambda b: (b, 0)),
            pl.BlockSpec((1, 1, _B), lambda b: (b, 0, 0)),
        ],
        out_specs=pl.BlockSpec((_SPAD, 2 * _F), lambda b: (0, 0)),
        out_shape=jax.ShapeDtypeStruct((_SPAD, 2 * _F), jnp.bfloat16),
        scratch_shapes=[
            pltpu.VMEM((_SPAD, _F), jnp.float32),
            pltpu.VMEM((_SPAD, _F), jnp.float32),
            pltpu.VMEM((_SPAD, 1), jnp.float32),
        ],
    )(x, seg3)

    out = pl.pallas_call(
        _norm_kernel,
        grid=(_NB,),
        in_specs=[
            pl.BlockSpec((_B, _F), lambda b: (b, 0)),
            pl.BlockSpec((1, 1, _B), lambda b: (b, 0, 0)),
            pl.BlockSpec((_SPAD, 2 * _F), lambda b: (0, 0)),
        ],
        out_specs=pl.BlockSpec((_B, _F), lambda b: (b, 0)),
        out_shape=jax.ShapeDtypeStruct((_N, _F), jnp.float32),
    )(x, seg3, table)

    return out
